# Initial kernel scaffold; baseline (speedup 1.0000x reference)
import jax
import jax.numpy as jnp
from jax.experimental import pallas as pl


def kernel(pos, rgb, opacity, w2c_r, w2c_t):
    raise NotImplementedError("write your pallas kernel here")


# speedup vs baseline: 19.5431x; 19.5431x over previous
import functools

import jax
import jax.numpy as jnp
from jax import lax
from jax.experimental import pallas as pl
from jax.experimental.pallas import tpu as pltpu
from jax.experimental.pallas import tpu_sc as plsc

N = 1000000
W = 1024.0
H = 1024.0
TILE = 16
NTX = 64
NTY = 64
NTILES = NTX * NTY
FX = 1000.0
FY = 1000.0
CX = 512.0
CY = 512.0
NEAR = 0.3

NC = 2
NS = 16
NW = NC * NS
P = 31744
N_PAD = NW * P
M = N_PAD // 1024
BM = 32
RW = P // 1024


def _bf16_bits(x):
    b16 = lax.bitcast_convert_type(x.astype(jnp.bfloat16), jnp.uint16)
    return b16.astype(jnp.uint32)


def _project_body(in_ref, out_ref):
    cam0 = in_ref[0]
    cam1 = in_ref[1]
    cam2 = in_ref[2]
    valid = cam2 > NEAR
    zs = jnp.where(valid, cam2, 1.0)
    u = cam0 / zs * FX + CX
    v = cam1 / zs * FY + CY
    inb = valid & (u >= 0.0) & (u < W) & (v >= 0.0) & (v < H)
    tx = jnp.clip(jnp.floor(u * (1.0 / TILE)).astype(jnp.int32), 0, NTX - 1)
    ty = jnp.clip(jnp.floor(v * (1.0 / TILE)).astype(jnp.int32), 0, NTY - 1)
    tid = jnp.where(inb, ty * NTX + tx, 0)
    w = jnp.where(inb, jax.nn.sigmoid(in_ref[6]), 0.0)
    i32 = lambda x: lax.bitcast_convert_type(x, jnp.int32)
    out_ref[:, 0, :] = i32(tid.astype(jnp.uint32) | (_bf16_bits(w) << 16))
    out_ref[:, 1, :] = i32(_bf16_bits(jax.nn.sigmoid(in_ref[3]) * w)
                           | (_bf16_bits(jax.nn.sigmoid(in_ref[4]) * w) << 16))
    out_ref[:, 2, :] = i32(_bf16_bits(jax.nn.sigmoid(in_ref[5]) * w))


def _project(ins):
    return pl.pallas_call(
        _project_body,
        grid=(M // BM,),
        in_specs=[pl.BlockSpec((7, BM, 1024), lambda i: (0, i, 0))],
        out_specs=pl.BlockSpec((BM, 3, 1024), lambda i: (i, 0, 0)),
        out_shape=jax.ShapeDtypeStruct((M, 3, 1024), jnp.int32),
    )(ins)


def _sc_bin_body(a_hbm, out_hbm, buf0, buf1, acc, sem0, sem1):
    wid = lax.axis_index("s") * NC + lax.axis_index("c")
    bufs = (buf0, buf1)
    sems = (sem0, sem1)

    def zero_body(i, carry):
        acc[pl.ds(i * 16, 16)] = jnp.zeros((16,), jnp.float32)
        return carry

    lax.fori_loop(0, 4 * NTILES // 16, zero_body, 0)

    def copy(k, b):
        return pltpu.make_async_copy(a_hbm.at[wid * RW + k], bufs[b], sems[b])

    copy(0, 0).start()
    copy(1, 1).start()

    def process(k, b):
        copy(k, b).wait()
        buf = bufs[b]

        def vec_body(j, c2):
            o = j * 16
            hi_mask = jnp.full((16,), -65536, jnp.int32)
            p0 = buf[pl.ds(o, 16)]
            p1 = buf[pl.ds(1024 + o, 16)]
            p2 = buf[pl.ds(2048 + o, 16)]
            idx = p0 & 0xFFF
            w = plsc.bitcast(p0 & hi_mask, jnp.float32)
            vr = plsc.bitcast(p1 << 16, jnp.float32)
            vg = plsc.bitcast(p1 & hi_mask, jnp.float32)
            vb = plsc.bitcast(p2 << 16, jnp.float32)
            plsc.addupdate_scatter(acc, [idx], vr)
            plsc.addupdate_scatter(acc, [idx + NTILES], vg)
            plsc.addupdate_scatter(acc, [idx + 2 * NTILES], vb)
            plsc.addupdate_scatter(acc, [idx + 3 * NTILES], w)
            return c2

        lax.fori_loop(0, 64, vec_body, 0)

    def chunk_body(k0, carry):
        for b in range(2):
            k = k0 * 2 + b
            process(k, b)

            @pl.when(k + 2 < RW)
            def _():
                copy(k + 2, b).start()
        return carry

    lax.fori_loop(0, RW // 2, chunk_body, 0)
    process(RW - 1, 0)
    pltpu.sync_copy(acc, out_hbm.at[wid])


@functools.cache
def _sc_bin():
    return pl.kernel(
        _sc_bin_body,
        out_type=jax.ShapeDtypeStruct((NW, 4 * NTILES), jnp.float32),
        mesh=plsc.VectorSubcoreMesh(core_axis_name="c", subcore_axis_name="s",
                                    num_cores=NC, num_subcores=NS),
        compiler_params=pltpu.CompilerParams(needs_layout_passes=False),
        scratch_types=[
            pltpu.VMEM((3072,), jnp.int32),
            pltpu.VMEM((3072,), jnp.int32),
            pltpu.VMEM((4 * NTILES,), jnp.float32),
            pltpu.SemaphoreType.DMA,
            pltpu.SemaphoreType.DMA,
        ],
    )


def _reduce_body(p_ref, out_ref):
    s = jnp.sum(p_ref[...], axis=0)
    den = s[3]
    out_ref[...] = jnp.clip(s[0:3] / (den + 1e-6), 0.0, 1.0)


def _reduce(partials):
    return pl.pallas_call(
        _reduce_body,
        out_shape=jax.ShapeDtypeStruct((3, NTILES), jnp.float32),
    )(partials)


def kernel(pos, rgb, opacity, w2c_r, w2c_t):
    pad = N_PAD - N
    cam = pos @ w2c_r.T + w2c_t
    ins = jnp.concatenate([cam.T, rgb.T, opacity[None]], axis=0)
    ins = jnp.pad(ins, ((0, 0), (0, pad))).reshape(7, M, 1024)

    a = _project(ins)
    partials = _sc_bin()(a.reshape(M, 3072))
    img = _reduce(partials.reshape(NW, 4, NTILES))
    return img.T


# speedup vs baseline: 20.7872x; 1.0637x over previous
import functools

import jax
import jax.numpy as jnp
from jax import lax
from jax.experimental import pallas as pl
from jax.experimental.pallas import tpu as pltpu
from jax.experimental.pallas import tpu_sc as plsc

N = 1000000
W = 1024.0
H = 1024.0
TILE = 16
NTX = 64
NTY = 64
NTILES = NTX * NTY
FX = 1000.0
FY = 1000.0
CX = 512.0
CY = 512.0
NEAR = 0.3

NC = 2
NS = 16
NW = NC * NS
P = 31744
N_PAD = NW * P
M = N_PAD // 1024
BM = 32
RW = P // 1024


def _bf16_bits(x):
    b16 = lax.bitcast_convert_type(x.astype(jnp.bfloat16), jnp.uint16)
    return b16.astype(jnp.uint32)


def _project_body(in_ref, out_ref):
    cam0 = in_ref[0]
    cam1 = in_ref[1]
    cam2 = in_ref[2]
    valid = cam2 > NEAR
    zs = jnp.where(valid, cam2, 1.0)
    u = cam0 / zs * FX + CX
    v = cam1 / zs * FY + CY
    inb = valid & (u >= 0.0) & (u < W) & (v >= 0.0) & (v < H)
    tx = jnp.clip(jnp.floor(u * (1.0 / TILE)).astype(jnp.int32), 0, NTX - 1)
    ty = jnp.clip(jnp.floor(v * (1.0 / TILE)).astype(jnp.int32), 0, NTY - 1)
    tid = jnp.where(inb, ty * NTX + tx, 0)
    w = jnp.where(inb, jax.nn.sigmoid(in_ref[6]), 0.0)
    i32 = lambda x: lax.bitcast_convert_type(x, jnp.int32)
    w0 = i32(tid.astype(jnp.uint32) | (_bf16_bits(w) << 16))
    w1 = i32(_bf16_bits(jax.nn.sigmoid(in_ref[3]) * w)
             | (_bf16_bits(jax.nn.sigmoid(in_ref[4]) * w) << 16))
    w2 = i32(_bf16_bits(jax.nn.sigmoid(in_ref[5]) * w))
    arr = jnp.stack([w0, w1, w2], axis=1)
    out_ref[...] = arr.reshape(3 * BM, 8, 128)


def _project(ins):
    return pl.pallas_call(
        _project_body,
        grid=(M // BM,),
        in_specs=[pl.BlockSpec((7, BM, 1024), lambda i: (0, i, 0))],
        out_specs=pl.BlockSpec((3 * BM, 8, 128), lambda i: (i, 0, 0)),
        out_shape=jax.ShapeDtypeStruct((3 * M, 8, 128), jnp.int32),
    )(ins)


def _sc_bin_body(a_hbm, out_hbm, buf0, buf1, acc, sem0, sem1):
    wid = lax.axis_index("s") * NC + lax.axis_index("c")
    bufs = (buf0, buf1)
    sems = (sem0, sem1)

    def zero_body(i, carry):
        acc[pl.ds(i * 16, 16)] = jnp.zeros((16,), jnp.float32)
        return carry

    lax.fori_loop(0, 4 * NTILES // 16, zero_body, 0)

    def copy(k, b):
        return pltpu.make_async_copy(a_hbm.at[pl.ds(3 * (wid * RW + k), 3)],
                                     bufs[b], sems[b])

    copy(0, 0).start()
    copy(1, 1).start()

    def process(k, b):
        copy(k, b).wait()
        buf = bufs[b]

        def vec_body(j, c2):
            r = j >> 3
            o = (j & 7) * 16
            hi_mask = jnp.full((16,), -65536, jnp.int32)
            p0 = buf[0, r, pl.ds(o, 16)]
            p1 = buf[1, r, pl.ds(o, 16)]
            p2 = buf[2, r, pl.ds(o, 16)]
            idx = p0 & 0xFFF
            w = plsc.bitcast(p0 & hi_mask, jnp.float32)
            vr = plsc.bitcast(p1 << 16, jnp.float32)
            vg = plsc.bitcast(p1 & hi_mask, jnp.float32)
            vb = plsc.bitcast(p2 << 16, jnp.float32)
            plsc.addupdate_scatter(acc, [idx], vr)
            plsc.addupdate_scatter(acc, [idx + NTILES], vg)
            plsc.addupdate_scatter(acc, [idx + 2 * NTILES], vb)
            plsc.addupdate_scatter(acc, [idx + 3 * NTILES], w)
            return c2

        lax.fori_loop(0, 64, vec_body, 0)

    def chunk_body(k0, carry):
        for b in range(2):
            k = k0 * 2 + b
            process(k, b)

            @pl.when(k + 2 < RW)
            def _():
                copy(k + 2, b).start()
        return carry

    lax.fori_loop(0, RW // 2, chunk_body, 0)
    process(RW - 1, 0)
    pltpu.sync_copy(acc, out_hbm.at[wid])


@functools.cache
def _sc_bin():
    return pl.kernel(
        _sc_bin_body,
        out_type=jax.ShapeDtypeStruct((NW, 4 * NTILES), jnp.float32),
        mesh=plsc.VectorSubcoreMesh(core_axis_name="c", subcore_axis_name="s",
                                    num_cores=NC, num_subcores=NS),
        compiler_params=pltpu.CompilerParams(needs_layout_passes=False),
        scratch_types=[
            pltpu.VMEM((3, 8, 128), jnp.int32),
            pltpu.VMEM((3, 8, 128), jnp.int32),
            pltpu.VMEM((4 * NTILES,), jnp.float32),
            pltpu.SemaphoreType.DMA,
            pltpu.SemaphoreType.DMA,
        ],
    )


def _reduce_body(p_ref, out_ref):
    s = jnp.sum(p_ref[...], axis=0)
    den = s[3]
    out_ref[...] = jnp.clip(s[0:3] / (den + 1e-6), 0.0, 1.0)


def _reduce(partials):
    return pl.pallas_call(
        _reduce_body,
        out_shape=jax.ShapeDtypeStruct((3, NTILES), jnp.float32),
    )(partials)


def kernel(pos, rgb, opacity, w2c_r, w2c_t):
    pad = N_PAD - N
    cam = pos @ w2c_r.T + w2c_t
    ins = jnp.concatenate([cam.T, rgb.T, opacity[None]], axis=0)
    ins = jnp.pad(ins, ((0, 0), (0, pad))).reshape(7, M, 1024)

    a = _project(ins)
    partials = _sc_bin()(a)
    img = _reduce(partials.reshape(NW, 4, NTILES))
    return img.T


# speedup vs baseline: 21.1951x; 1.0196x over previous
import functools

import jax
import jax.numpy as jnp
from jax import lax
from jax.experimental import pallas as pl
from jax.experimental.pallas import tpu as pltpu
from jax.experimental.pallas import tpu_sc as plsc

N = 1000000
W = 1024.0
H = 1024.0
TILE = 16
NTX = 64
NTY = 64
NTILES = NTX * NTY
FX = 1000.0
FY = 1000.0
CX = 512.0
CY = 512.0
NEAR = 0.3

NC = 2
NS = 16
NW = NC * NS
P = 32768
N_PAD = NW * P
M = N_PAD // 1024
BM = 32
RW = P // 1024


def _bf16_bits(x):
    b16 = lax.bitcast_convert_type(x.astype(jnp.bfloat16), jnp.uint16)
    return b16.astype(jnp.uint32)


def _project_body(in_ref, out_ref):
    cam0 = in_ref[0]
    cam1 = in_ref[1]
    cam2 = in_ref[2]
    valid = cam2 > NEAR
    zs = jnp.where(valid, cam2, 1.0)
    u = cam0 / zs * FX + CX
    v = cam1 / zs * FY + CY
    inb = valid & (u >= 0.0) & (u < W) & (v >= 0.0) & (v < H)
    tx = jnp.clip(jnp.floor(u * (1.0 / TILE)).astype(jnp.int32), 0, NTX - 1)
    ty = jnp.clip(jnp.floor(v * (1.0 / TILE)).astype(jnp.int32), 0, NTY - 1)
    tid = jnp.where(inb, ty * NTX + tx, 0)
    w = jnp.where(inb, jax.nn.sigmoid(in_ref[6]), 0.0)
    f32 = lambda x: lax.bitcast_convert_type(x, jnp.float32)
    out_ref[:, 0, :] = f32(tid.astype(jnp.uint32) | (_bf16_bits(w) << 16))
    out_ref[:, 1, :] = f32(_bf16_bits(jax.nn.sigmoid(in_ref[3]) * w)
                           | (_bf16_bits(jax.nn.sigmoid(in_ref[4]) * w) << 16))
    out_ref[:, 2, :] = f32(_bf16_bits(jax.nn.sigmoid(in_ref[5]) * w))


def _project(ins):
    return pl.pallas_call(
        _project_body,
        grid=(M // BM,),
        in_specs=[pl.BlockSpec((7, BM, 1024), lambda i: (0, i, 0))],
        out_specs=pl.BlockSpec((BM, 3, 1024), lambda i: (i, 0, 0)),
        out_shape=jax.ShapeDtypeStruct((M, 3, 1024), jnp.float32),
    )(ins)


R0C = 16
R1C = RW - R0C


def _sc_bin_body(a_hbm, out_hbm, buf0, buf1, acc, sem0, sem1):
    wid = lax.axis_index("s") * NC + lax.axis_index("c")
    base = wid * RW

    def zero_body(i, carry):
        acc[pl.ds(i * 16, 16)] = jnp.zeros((16,), jnp.float32)
        return carry

    lax.fori_loop(0, 4 * NTILES // 16, zero_body, 0)

    c0 = pltpu.make_async_copy(a_hbm.at[pl.ds(base, R0C)], buf0, sem0)
    c1 = pltpu.make_async_copy(a_hbm.at[pl.ds(base + R0C, R1C)], buf1, sem1)
    c0.start()
    c1.start()

    def process(buf, nrows):
        def vec_body(j, c2):
            r = j >> 6
            o = (j & 63) * 16
            hi_mask = jnp.full((16,), -65536, jnp.int32)
            p0 = plsc.bitcast(buf[r, pl.ds(o, 16)], jnp.int32)
            p1 = plsc.bitcast(buf[r, pl.ds(1024 + o, 16)], jnp.int32)
            p2 = plsc.bitcast(buf[r, pl.ds(2048 + o, 16)], jnp.int32)
            idx = p0 & 0xFFF
            w = plsc.bitcast(p0 & hi_mask, jnp.float32)
            vr = plsc.bitcast(p1 << 16, jnp.float32)
            vg = plsc.bitcast(p1 & hi_mask, jnp.float32)
            vb = plsc.bitcast(p2 << 16, jnp.float32)
            plsc.addupdate_scatter(acc, [idx], vr)
            plsc.addupdate_scatter(acc, [idx + NTILES], vg)
            plsc.addupdate_scatter(acc, [idx + 2 * NTILES], vb)
            plsc.addupdate_scatter(acc, [idx + 3 * NTILES], w)
            return c2

        lax.fori_loop(0, nrows * 64, vec_body, 0)

    c0.wait()
    process(buf0, R0C)
    c1.wait()
    process(buf1, R1C)
    pltpu.sync_copy(acc, out_hbm.at[wid])


@functools.cache
def _sc_bin():
    return pl.kernel(
        _sc_bin_body,
        out_type=jax.ShapeDtypeStruct((NW, 4 * NTILES), jnp.float32),
        mesh=plsc.VectorSubcoreMesh(core_axis_name="c", subcore_axis_name="s",
                                    num_cores=NC, num_subcores=NS),
        compiler_params=pltpu.CompilerParams(needs_layout_passes=False),
        scratch_types=[
            pltpu.VMEM((R0C, 3072), jnp.float32),
            pltpu.VMEM((R1C, 3072), jnp.float32),
            pltpu.VMEM((4 * NTILES,), jnp.float32),
            pltpu.SemaphoreType.DMA,
            pltpu.SemaphoreType.DMA,
        ],
    )


def _reduce_body(p_ref, out_ref):
    s = jnp.sum(p_ref[...], axis=0)
    den = s[3]
    out_ref[...] = jnp.clip(s[0:3] / (den + 1e-6), 0.0, 1.0)


def _reduce(partials):
    return pl.pallas_call(
        _reduce_body,
        out_shape=jax.ShapeDtypeStruct((3, NTILES), jnp.float32),
    )(partials)


def kernel(pos, rgb, opacity, w2c_r, w2c_t):
    pad = N_PAD - N
    cam = pos @ w2c_r.T + w2c_t
    ins = jnp.concatenate([cam.T, rgb.T, opacity[None]], axis=0)
    ins = jnp.pad(ins, ((0, 0), (0, pad))).reshape(7, M, 1024)

    a = _project(ins)
    partials = _sc_bin()(a.reshape(M, 3072))
    img = _reduce(partials.reshape(NW, 4, NTILES))
    return img.T


# speedup vs baseline: 21.5149x; 1.0151x over previous
import functools

import jax
import jax.numpy as jnp
from jax import lax
from jax.experimental import pallas as pl
from jax.experimental.pallas import tpu as pltpu
from jax.experimental.pallas import tpu_sc as plsc

N = 1000000
W = 1024.0
H = 1024.0
TILE = 16
NTX = 64
NTY = 64
NTILES = NTX * NTY
FX = 1000.0
FY = 1000.0
CX = 512.0
CY = 512.0
NEAR = 0.3

NC = 2
NS = 16
NW = NC * NS
P = 32768
N_PAD = NW * P
M = N_PAD // 1024
BM = 32
RW = P // 1024


def _bf16_bits(x):
    b16 = lax.bitcast_convert_type(x.astype(jnp.bfloat16), jnp.uint16)
    return b16.astype(jnp.uint32)


def _project_body(in_ref, out_ref):
    cam0 = in_ref[0]
    cam1 = in_ref[1]
    cam2 = in_ref[2]
    valid = cam2 > NEAR
    zs = jnp.where(valid, cam2, 1.0)
    u = cam0 / zs * FX + CX
    v = cam1 / zs * FY + CY
    inb = valid & (u >= 0.0) & (u < W) & (v >= 0.0) & (v < H)
    tx = jnp.clip(jnp.floor(u * (1.0 / TILE)).astype(jnp.int32), 0, NTX - 1)
    ty = jnp.clip(jnp.floor(v * (1.0 / TILE)).astype(jnp.int32), 0, NTY - 1)
    tid = jnp.where(inb, ty * NTX + tx, 0)
    w = jnp.where(inb, jax.nn.sigmoid(in_ref[6]), 0.0)
    f32 = lambda x: lax.bitcast_convert_type(x, jnp.float32)
    out_ref[:, 0, :] = f32(tid.astype(jnp.uint32) | (_bf16_bits(w) << 16))
    out_ref[:, 1, :] = f32(_bf16_bits(jax.nn.sigmoid(in_ref[3]) * w)
                           | (_bf16_bits(jax.nn.sigmoid(in_ref[4]) * w) << 16))
    out_ref[:, 2, :] = f32(_bf16_bits(jax.nn.sigmoid(in_ref[5]) * w))


def _project(ins):
    return pl.pallas_call(
        _project_body,
        grid=(M // BM,),
        in_specs=[pl.BlockSpec((7, BM, 1024), lambda i: (0, i, 0))],
        out_specs=pl.BlockSpec((BM, 3, 1024), lambda i: (i, 0, 0)),
        out_shape=jax.ShapeDtypeStruct((M, 3, 1024), jnp.float32),
    )(ins)


NBUF = 4


def _sc_bin_body(a_hbm, out_hbm, buf0, buf1, buf2, buf3, acc,
                 sem0, sem1, sem2, sem3):
    wid = lax.axis_index("s") * NC + lax.axis_index("c")
    bufs = (buf0, buf1, buf2, buf3)
    sems = (sem0, sem1, sem2, sem3)

    def zero_body(i, carry):
        acc[pl.ds(i * 16, 16)] = jnp.zeros((16,), jnp.float32)
        return carry

    lax.fori_loop(0, 4 * NTILES // 16, zero_body, 0)

    def copy(k, b):
        return pltpu.make_async_copy(a_hbm.at[wid * RW + k], bufs[b], sems[b])

    for b in range(NBUF):
        copy(b, b).start()

    def process(k, b):
        copy(k, b).wait()
        buf = bufs[b]

        def vec_body(j, c2):
            o = j * 16
            hi_mask = jnp.full((16,), -65536, jnp.int32)
            p0 = plsc.bitcast(buf[pl.ds(o, 16)], jnp.int32)
            p1 = plsc.bitcast(buf[pl.ds(1024 + o, 16)], jnp.int32)
            p2 = plsc.bitcast(buf[pl.ds(2048 + o, 16)], jnp.int32)
            idx = p0 & 0xFFF
            w = plsc.bitcast(p0 & hi_mask, jnp.float32)
            vr = plsc.bitcast(p1 << 16, jnp.float32)
            vg = plsc.bitcast(p1 & hi_mask, jnp.float32)
            vb = plsc.bitcast(p2 << 16, jnp.float32)
            plsc.addupdate_scatter(acc, [idx], vr)
            plsc.addupdate_scatter(acc, [idx + NTILES], vg)
            plsc.addupdate_scatter(acc, [idx + 2 * NTILES], vb)
            plsc.addupdate_scatter(acc, [idx + 3 * NTILES], w)
            return c2

        lax.fori_loop(0, 64, vec_body, 0)

    def chunk_body(k0, carry):
        for b in range(NBUF):
            k = k0 * NBUF + b
            process(k, b)

            @pl.when(k + NBUF < RW)
            def _():
                copy(k + NBUF, b).start()
        return carry

    lax.fori_loop(0, RW // NBUF, chunk_body, 0)
    pltpu.sync_copy(acc, out_hbm.at[wid])


@functools.cache
def _sc_bin():
    return pl.kernel(
        _sc_bin_body,
        out_type=jax.ShapeDtypeStruct((NW, 4 * NTILES), jnp.float32),
        mesh=plsc.VectorSubcoreMesh(core_axis_name="c", subcore_axis_name="s",
                                    num_cores=NC, num_subcores=NS),
        compiler_params=pltpu.CompilerParams(needs_layout_passes=False),
        scratch_types=[
            pltpu.VMEM((3072,), jnp.float32),
            pltpu.VMEM((3072,), jnp.float32),
            pltpu.VMEM((3072,), jnp.float32),
            pltpu.VMEM((3072,), jnp.float32),
            pltpu.VMEM((4 * NTILES,), jnp.float32),
            pltpu.SemaphoreType.DMA,
            pltpu.SemaphoreType.DMA,
            pltpu.SemaphoreType.DMA,
            pltpu.SemaphoreType.DMA,
        ],
    )


def _reduce_body(p_ref, out_ref):
    s = jnp.sum(p_ref[...], axis=0)
    den = s[3]
    out_ref[...] = jnp.clip(s[0:3] / (den + 1e-6), 0.0, 1.0)


def _reduce(partials):
    return pl.pallas_call(
        _reduce_body,
        out_shape=jax.ShapeDtypeStruct((3, NTILES), jnp.float32),
    )(partials)


def kernel(pos, rgb, opacity, w2c_r, w2c_t):
    pad = N_PAD - N
    cam = pos @ w2c_r.T + w2c_t
    ins = jnp.concatenate([cam.T, rgb.T, opacity[None]], axis=0)
    ins = jnp.pad(ins, ((0, 0), (0, pad))).reshape(7, M, 1024)

    a = _project(ins)
    partials = _sc_bin()(a.reshape(M, 3072))
    img = _reduce(partials.reshape(NW, 4, NTILES))
    return img.T


# speedup vs baseline: 23.7994x; 1.1062x over previous
import functools

import jax
import jax.numpy as jnp
from jax import lax
from jax.experimental import pallas as pl
from jax.experimental.pallas import tpu as pltpu
from jax.experimental.pallas import tpu_sc as plsc

N = 1000000
W = 1024.0
H = 1024.0
TILE = 16
NTX = 64
NTY = 64
NTILES = NTX * NTY
FX = 1000.0
FY = 1000.0
CX = 512.0
CY = 512.0
NEAR = 0.3

NC = 2
NS = 16
NW = NC * NS
P = 31744
N_PAD = NW * P
M = N_PAD // 1024
BM = 32
RW = P // 1024


def _project_body(cam_ref, rgb_ref, op_ref, out_ref):
    cam0 = cam_ref[0]
    cam1 = cam_ref[1]
    cam2 = cam_ref[2]
    valid = cam2 > NEAR
    zs = jnp.where(valid, cam2, 1.0)
    u = cam0 / zs * FX + CX
    v = cam1 / zs * FY + CY
    inb = valid & (u >= 0.0) & (u < W) & (v >= 0.0) & (v < H)
    tx = jnp.clip(jnp.floor(u * (1.0 / TILE)).astype(jnp.int32), 0, NTX - 1)
    ty = jnp.clip(jnp.floor(v * (1.0 / TILE)).astype(jnp.int32), 0, NTY - 1)
    tid = jnp.where(inb, ty * NTX + tx, 0)
    w = jnp.where(inb, jax.nn.sigmoid(op_ref[...]), 0.0)
    out_ref[:, 0, :] = lax.bitcast_convert_type(tid, jnp.float32)
    out_ref[:, 1, :] = jax.nn.sigmoid(rgb_ref[0]) * w
    out_ref[:, 2, :] = jax.nn.sigmoid(rgb_ref[1]) * w
    out_ref[:, 3, :] = jax.nn.sigmoid(rgb_ref[2]) * w
    out_ref[:, 4, :] = w


def _project(cam3, rgb3, op2):
    return pl.pallas_call(
        _project_body,
        grid=(M // BM,),
        in_specs=[
            pl.BlockSpec((3, BM, 1024), lambda i: (0, i, 0)),
            pl.BlockSpec((3, BM, 1024), lambda i: (0, i, 0)),
            pl.BlockSpec((BM, 1024), lambda i: (i, 0)),
        ],
        out_specs=pl.BlockSpec((BM, 5, 1024), lambda i: (i, 0, 0)),
        out_shape=jax.ShapeDtypeStruct((M, 5, 1024), jnp.float32),
    )(cam3, rgb3, op2)


def _sc_bin_body(a_hbm, out_hbm, buf0, buf1, acc, sem0, sem1):
    wid = lax.axis_index("s") * NC + lax.axis_index("c")
    bufs = (buf0, buf1)
    sems = (sem0, sem1)

    def zero_body(i, carry):
        acc[pl.ds(i * 16, 16)] = jnp.zeros((16,), jnp.float32)
        return carry

    lax.fori_loop(0, 4 * NTILES // 16, zero_body, 0)

    def copy(k, b):
        return pltpu.make_async_copy(a_hbm.at[wid * RW + k], bufs[b], sems[b])

    copy(0, 0).start()
    copy(1, 1).start()

    def process(k, b):
        copy(k, b).wait()
        buf = bufs[b]

        def vec_body(j, c2):
            o = j * 16
            idx = plsc.bitcast(buf[pl.ds(o, 16)], jnp.int32)
            plsc.addupdate_scatter(acc, [idx], buf[pl.ds(1024 + o, 16)])
            plsc.addupdate_scatter(acc, [idx + NTILES], buf[pl.ds(2048 + o, 16)])
            plsc.addupdate_scatter(acc, [idx + 2 * NTILES], buf[pl.ds(3072 + o, 16)])
            plsc.addupdate_scatter(acc, [idx + 3 * NTILES], buf[pl.ds(4096 + o, 16)])
            return c2

        lax.fori_loop(0, 64, vec_body, 0)

    def chunk_body(k0, carry):
        for b in range(2):
            k = k0 * 2 + b
            process(k, b)

            @pl.when(k + 2 < RW)
            def _():
                copy(k + 2, b).start()
        return carry

    lax.fori_loop(0, RW // 2, chunk_body, 0)
    process(RW - 1, 0)
    pltpu.sync_copy(acc, out_hbm.at[wid])


@functools.cache
def _sc_bin():
    return pl.kernel(
        _sc_bin_body,
        out_type=jax.ShapeDtypeStruct((NW, 4 * NTILES), jnp.float32),
        mesh=plsc.VectorSubcoreMesh(core_axis_name="c", subcore_axis_name="s",
                                    num_cores=NC, num_subcores=NS),
        compiler_params=pltpu.CompilerParams(needs_layout_passes=False),
        scratch_types=[
            pltpu.VMEM((5120,), jnp.float32),
            pltpu.VMEM((5120,), jnp.float32),
            pltpu.VMEM((4 * NTILES,), jnp.float32),
            pltpu.SemaphoreType.DMA,
            pltpu.SemaphoreType.DMA,
        ],
    )


def _reduce_body(p_ref, out_ref):
    s = jnp.sum(p_ref[...], axis=0)
    den = s[3]
    out_ref[...] = jnp.clip(s[0:3] / (den + 1e-6), 0.0, 1.0)


def _reduce(partials):
    return pl.pallas_call(
        _reduce_body,
        out_shape=jax.ShapeDtypeStruct((3, NTILES), jnp.float32),
    )(partials)


def kernel(pos, rgb, opacity, w2c_r, w2c_t):
    pad = N_PAD - N
    cam = pos @ w2c_r.T + w2c_t
    cam3 = jnp.pad(cam.T, ((0, 0), (0, pad))).reshape(3, M, 1024)
    rgb3 = jnp.pad(rgb.T, ((0, 0), (0, pad))).reshape(3, M, 1024)
    op2 = jnp.pad(opacity, (0, pad)).reshape(M, 1024)

    a = _project(cam3, rgb3, op2)
    partials = _sc_bin()(a.reshape(M, 5120))
    img = _reduce(partials.reshape(NW, 4, NTILES))
    return img.T
